# baseline (speedup 1.0000x reference)
import functools

import jax
import jax.numpy as jnp
from jax import lax
from jax.experimental import pallas as pl
from jax.experimental.pallas import tpu as pltpu
from jax.experimental.pallas import tpu_sc as plsc

VOCAB = 21128
DIM = 768
SEQ = 1000
PAD_ID = 0

_NUM_WORKERS = 32
_ROWS_PER_WORKER = 32
_LAST_BASE = SEQ - _ROWS_PER_WORKER


def _gather_body(tok_hbm, table_hbm, out_hbm, idx_v, rows_v, sem):
    wid = lax.axis_index("s") * 2 + lax.axis_index("c")
    base = jnp.minimum(wid * _ROWS_PER_WORKER, _LAST_BASE)
    pltpu.sync_copy(tok_hbm.at[pl.ds(base, _ROWS_PER_WORKER)], idx_v)
    pltpu.async_copy(table_hbm.at[idx_v], rows_v, sem).wait()
    pltpu.sync_copy(rows_v, out_hbm.at[pl.ds(base, _ROWS_PER_WORKER)])


@jax.jit
def _embed(tokenids, table):
    mesh = plsc.VectorSubcoreMesh(core_axis_name="c", subcore_axis_name="s")
    run = pl.kernel(
        _gather_body,
        out_type=jax.ShapeDtypeStruct((SEQ, DIM), jnp.float32),
        mesh=mesh,
        scratch_types=[
            pltpu.VMEM((_ROWS_PER_WORKER,), jnp.int32),
            pltpu.VMEM((_ROWS_PER_WORKER, DIM), jnp.float32),
            pltpu.SemaphoreType.DMA,
        ],
    )
    return run(tokenids, table)


def kernel(tokenids, table):
    token_ebd = _embed(tokenids, table)
    pad = jnp.array([PAD_ID], dtype=tokenids.dtype)
    labels = jnp.concatenate((tokenids[1:], pad))
    return (token_ebd, labels)
